# baseline (speedup 1.0000x reference)
import jax
import jax.numpy as jnp
from jax.experimental import pallas as pl

K = 8
N_EXPERTS = 64
BLOCK_T = 1024


CHUNK = 512


def _topk8(scores):
    lane = jax.lax.broadcasted_iota(jnp.int32, scores.shape, 1)
    neg_inf = jnp.float32(-jnp.inf)
    vals = []
    idxs = []
    s = scores
    for _ in range(K):
        v = jnp.max(s, axis=-1)
        i = jnp.argmax(s, axis=-1)
        vals.append(v)
        idxs.append(i.astype(jnp.int32))
        s = jnp.where(lane == i[:, None], neg_inf, s)
    return jnp.stack(idxs, axis=-1), jnp.stack(vals, axis=-1)


def _router_kernel(x_ref, w_ref, idx_ref, val_ref):
    w = w_ref[...]
    for c in range(BLOCK_T // CHUNK):
        rows = slice(c * CHUNK, (c + 1) * CHUNK)
        scores = jax.lax.dot_general(
            x_ref[rows, :], w, (((1,), (1,)), ((), ())),
            preferred_element_type=jnp.float32)
        idx_c, val_c = _topk8(scores)
        idx_ref[rows, :] = idx_c
        val_ref[rows, :] = val_c


@jax.jit
def kernel(x, W):
    b, s_len, d = x.shape
    t = b * s_len
    xf = x.reshape(t, d)
    grid = (t // BLOCK_T,)
    idx, val = pl.pallas_call(
        _router_kernel,
        grid=grid,
        in_specs=[
            pl.BlockSpec((BLOCK_T, d), lambda i: (i, 0)),
            pl.BlockSpec((N_EXPERTS, d), lambda i: (0, 0)),
        ],
        out_specs=[
            pl.BlockSpec((BLOCK_T, K), lambda i: (i, 0)),
            pl.BlockSpec((BLOCK_T, K), lambda i: (i, 0)),
        ],
        out_shape=[
            jax.ShapeDtypeStruct((t, K), jnp.int32),
            jax.ShapeDtypeStruct((t, K), jnp.float32),
        ],
    )(xf, W)
    return idx.reshape(b, s_len, K), val.reshape(b, s_len, K)


# speedup vs baseline: 1.0482x; 1.0482x over previous
import jax
import jax.numpy as jnp
from jax.experimental import pallas as pl

K = 8
N_EXPERTS = 64
BLOCK_T = 1024


CHUNK = 256


def _topk8(scores):
    lane = jax.lax.broadcasted_iota(jnp.int32, scores.shape, 1)
    neg_inf = jnp.float32(-jnp.inf)
    vals = []
    idxs = []
    s = scores
    for _ in range(K):
        v = jnp.max(s, axis=-1)
        i = jnp.argmax(s, axis=-1)
        vals.append(v)
        idxs.append(i.astype(jnp.int32))
        s = jnp.where(lane == i[:, None], neg_inf, s)
    return jnp.stack(idxs, axis=-1), jnp.stack(vals, axis=-1)


def _router_kernel(x_ref, w_ref, idx_ref, val_ref):
    w = w_ref[...]
    for c in range(BLOCK_T // CHUNK):
        rows = slice(c * CHUNK, (c + 1) * CHUNK)
        scores = jax.lax.dot_general(
            x_ref[rows, :], w, (((1,), (1,)), ((), ())),
            preferred_element_type=jnp.float32)
        idx_c, val_c = _topk8(scores)
        idx_ref[rows, :] = idx_c
        val_ref[rows, :] = val_c


@jax.jit
def kernel(x, W):
    b, s_len, d = x.shape
    t = b * s_len
    xf = x.reshape(t, d)
    grid = (t // BLOCK_T,)
    idx, val = pl.pallas_call(
        _router_kernel,
        grid=grid,
        in_specs=[
            pl.BlockSpec((BLOCK_T, d), lambda i: (i, 0)),
            pl.BlockSpec((N_EXPERTS, d), lambda i: (0, 0)),
        ],
        out_specs=[
            pl.BlockSpec((BLOCK_T, K), lambda i: (i, 0)),
            pl.BlockSpec((BLOCK_T, K), lambda i: (i, 0)),
        ],
        out_shape=[
            jax.ShapeDtypeStruct((t, K), jnp.int32),
            jax.ShapeDtypeStruct((t, K), jnp.float32),
        ],
    )(xf, W)
    return idx.reshape(b, s_len, K), val.reshape(b, s_len, K)
